# Initial kernel scaffold; baseline (speedup 1.0000x reference)
import jax
import jax.numpy as jnp
from jax.experimental import pallas as pl


def kernel(x, edge_index, edge_attr, Win_w, Win_b, Wl, bl, Wr, gam, bet, E1w, E1b, E2w, E2b, E3w, E3b):
    raise NotImplementedError("write your pallas kernel here")


# speedup vs baseline: 2.6896x; 2.6896x over previous
import functools

import jax
import jax.numpy as jnp
from jax import lax
from jax.experimental import pallas as pl
from jax.experimental.pallas import tpu as pltpu
from jax.experimental.pallas import tpu_sc as plsc

_N = 10000
_NP = 10112
_E = 320000
_EP = 327680
_D = 128
_H = 64
_DE = 16
_NC = 2
_NS = 16
_K = 128
_EPW = _EP // (_NC * _NS)
_CH = _EPW // _K
_RPS = _NP // _NS


def _sc_mesh():
    return plsc.VectorSubcoreMesh(core_axis_name="c", subcore_axis_name="s")


_SC_PARAMS = pltpu.CompilerParams(use_tc_tiling_on_sc=False)


def _fill_const(ref, rows, width, value):
    v = jnp.full((16,), value, jnp.float32)

    def row(r, carry):
        def col(q, carry2):
            ref[r, pl.ds(q * 16, 16)] = v
            return carry2
        return lax.fori_loop(0, width // 16, col, carry)

    lax.fori_loop(0, rows, row, None)


def _zero_shared_slice(zsrc, shared, base, rows):
    full, rem = rows // _K, rows % _K
    for i in range(full):
        pltpu.sync_copy(zsrc, shared.at[pl.ds(base + i * _K, _K)])
    if rem:
        pltpu.sync_copy(zsrc.at[pl.ds(0, rem)], shared.at[pl.ds(base + full * _K, rem)])


def _build_sc_deg():
    @functools.partial(
        pl.kernel,
        out_type=jax.ShapeDtypeStruct((_NC, _NP, 16), jnp.float32),
        mesh=_sc_mesh(),
        compiler_params=_SC_PARAMS,
        scratch_types=[
            pltpu.VMEM((_K,), jnp.int32),
            pltpu.VMEM((_K, 16), jnp.float32),
            pltpu.VMEM((_K, 16), jnp.float32),
            pltpu.VMEM_SHARED((_NP, 16), jnp.float32),
        ],
    )
    def deg_kernel(dst_hbm, out_hbm, dst_v, ones_v, zer_v, deg_sh):
        c = lax.axis_index("c")
        s = lax.axis_index("s")
        wid = c * _NS + s
        _fill_const(ones_v, _K, 16, 1.0)
        _fill_const(zer_v, _K, 16, 0.0)
        base = s * _RPS
        _zero_shared_slice(zer_v, deg_sh, base, _RPS)
        plsc.subcore_barrier()
        ebase = wid * _EPW

        def chunk(j, carry):
            pltpu.sync_copy(dst_hbm.at[pl.ds(ebase + j * _K, _K)], dst_v)
            pltpu.sync_copy(ones_v, deg_sh.at[dst_v], add=True)
            return carry

        lax.fori_loop(0, _CH, chunk, None)
        plsc.subcore_barrier()
        pltpu.sync_copy(deg_sh.at[pl.ds(base, _RPS)], out_hbm.at[c, pl.ds(base, _RPS)])

    return deg_kernel


def _build_sc_agg():
    @functools.partial(
        pl.kernel,
        out_type=jax.ShapeDtypeStruct((_NC, _NP, _H), jnp.float32),
        mesh=_sc_mesh(),
        compiler_params=_SC_PARAMS,
        scratch_types=[
            pltpu.VMEM((_K,), jnp.int32),
            pltpu.VMEM((_K,), jnp.int32),
            pltpu.VMEM((_K, _H), jnp.float32),
            pltpu.VMEM((_K, _H), jnp.float32),
            pltpu.VMEM_SHARED((_NP, _H), jnp.float32),
            pltpu.SemaphoreType.DMA,
        ],
    )
    def agg_kernel(h_hbm, src_hbm, dst_hbm, out_hbm,
                   src_v, dst_v, rows_v, zer_v, agg_sh, sem):
        c = lax.axis_index("c")
        s = lax.axis_index("s")
        wid = c * _NS + s
        _fill_const(zer_v, _K, _H, 0.0)
        base = s * _RPS
        _zero_shared_slice(zer_v, agg_sh, base, _RPS)
        plsc.subcore_barrier()
        ebase = wid * _EPW

        def chunk(j, carry):
            o = ebase + j * _K
            pltpu.sync_copy(src_hbm.at[pl.ds(o, _K)], src_v)
            pltpu.sync_copy(dst_hbm.at[pl.ds(o, _K)], dst_v)
            pltpu.async_copy(h_hbm.at[src_v], rows_v, sem).wait()
            pltpu.sync_copy(rows_v, agg_sh.at[dst_v], add=True)
            return carry

        lax.fori_loop(0, _CH, chunk, None)
        plsc.subcore_barrier()
        pltpu.sync_copy(agg_sh.at[pl.ds(base, _RPS)], out_hbm.at[c, pl.ds(base, _RPS)])

    return agg_kernel


def _build_sc_edge_gather():
    @functools.partial(
        pl.kernel,
        out_type=(jax.ShapeDtypeStruct((_EP, _H), jnp.float32),
                  jax.ShapeDtypeStruct((_EP, _H), jnp.float32)),
        mesh=_sc_mesh(),
        compiler_params=_SC_PARAMS,
        scratch_types=[
            pltpu.VMEM((_K,), jnp.int32),
            pltpu.VMEM((_K,), jnp.int32),
            pltpu.VMEM((_K, _H), jnp.float32),
            pltpu.VMEM((_K, _H), jnp.float32),
            pltpu.SemaphoreType.DMA,
            pltpu.SemaphoreType.DMA,
        ],
    )
    def gather_kernel(a_hbm, b_hbm, src_hbm, dst_hbm, sa_hbm, sb_hbm,
                      src_v, dst_v, ra_v, rb_v, sem_a, sem_b):
        c = lax.axis_index("c")
        s = lax.axis_index("s")
        wid = c * _NS + s
        ebase = wid * _EPW

        def chunk(j, carry):
            o = ebase + j * _K
            pltpu.sync_copy(src_hbm.at[pl.ds(o, _K)], src_v)
            pltpu.sync_copy(dst_hbm.at[pl.ds(o, _K)], dst_v)
            da = pltpu.async_copy(a_hbm.at[src_v], ra_v, sem_a)
            db = pltpu.async_copy(b_hbm.at[dst_v], rb_v, sem_b)
            da.wait()
            db.wait()
            pltpu.sync_copy(ra_v, sa_hbm.at[pl.ds(o, _K)])
            pltpu.sync_copy(rb_v, sb_hbm.at[pl.ds(o, _K)])
            return carry

        lax.fori_loop(0, _CH, chunk, None)

    return gather_kernel


def _tc_in_body(x_ref, w_ref, b_ref, o_ref):
    o_ref[...] = (jnp.dot(x_ref[...], w_ref[...],
                          preferred_element_type=jnp.float32) + b_ref[...])


def _build_tc_in():
    return pl.pallas_call(
        _tc_in_body,
        out_shape=jax.ShapeDtypeStruct((_NP, _H), jnp.float32),
    )


def _tc_layer_body(aggp_ref, degp_ref, h_ref, wl_ref, bl_ref, wr_ref,
                   g_ref, be_ref, o_ref):
    agg = aggp_ref[0] + aggp_ref[1]
    d = degp_ref[0] + degp_ref[1]
    deg = jnp.maximum(d[:, 0:1], 1.0)
    aggm = agg / deg
    h = h_ref[...]
    t = (jnp.dot(aggm, wl_ref[...], preferred_element_type=jnp.float32)
         + bl_ref[...]
         + jnp.dot(h, wr_ref[...], preferred_element_type=jnp.float32))
    mu = jnp.mean(t, axis=-1, keepdims=True)
    var = jnp.mean((t - mu) ** 2, axis=-1, keepdims=True)
    hn = (t - mu) * lax.rsqrt(var + 1e-5) * g_ref[...] + be_ref[...]
    o_ref[...] = jnp.maximum(hn, 0.0)


def _build_tc_layer():
    return pl.pallas_call(
        _tc_layer_body,
        out_shape=jax.ShapeDtypeStruct((_NP, _H), jnp.float32),
    )


def _tc_ab_body(h_ref, wu_ref, wv_ref, a_ref, b_ref):
    h = h_ref[...]
    a_ref[...] = jnp.dot(h, wu_ref[...], preferred_element_type=jnp.float32)
    b_ref[...] = jnp.dot(h, wv_ref[...], preferred_element_type=jnp.float32)


def _build_tc_ab():
    return pl.pallas_call(
        _tc_ab_body,
        out_shape=(jax.ShapeDtypeStruct((_NP, _H), jnp.float32),
                   jax.ShapeDtypeStruct((_NP, _H), jnp.float32)),
    )


_EB = 10240


def _tc_edge_body(sa_ref, sb_ref, ea_ref, w1_ref, b1_ref, w2_ref, b2_ref,
                  w3_ref, b3_ref, o_ref):
    z1 = jnp.maximum(
        sa_ref[...] + sb_ref[...]
        + jnp.dot(ea_ref[...], w1_ref[...], preferred_element_type=jnp.float32)
        + b1_ref[...], 0.0)
    z2 = jnp.maximum(
        jnp.dot(z1, w2_ref[...], preferred_element_type=jnp.float32)
        + b2_ref[...], 0.0)
    o_ref[...] = jnp.sum(z2 * w3_ref[...], axis=1, keepdims=True) + b3_ref[...]


def _build_tc_edge():
    return pl.pallas_call(
        _tc_edge_body,
        grid=(_EP // _EB,),
        in_specs=[
            pl.BlockSpec((_EB, _H), lambda i: (i, 0)),
            pl.BlockSpec((_EB, _H), lambda i: (i, 0)),
            pl.BlockSpec((_EB, _DE), lambda i: (i, 0)),
            pl.BlockSpec((_DE, _H), lambda i: (0, 0)),
            pl.BlockSpec((1, _H), lambda i: (0, 0)),
            pl.BlockSpec((_H, 32), lambda i: (0, 0)),
            pl.BlockSpec((1, 32), lambda i: (0, 0)),
            pl.BlockSpec((1, 32), lambda i: (0, 0)),
            pl.BlockSpec((1, 1), lambda i: (0, 0)),
        ],
        out_specs=pl.BlockSpec((_EB, 1), lambda i: (i, 0)),
        out_shape=jax.ShapeDtypeStruct((_EP, 1), jnp.float32),
    )


_SC_DEG = _build_sc_deg()
_SC_AGG = _build_sc_agg()
_SC_EDGE_GATHER = _build_sc_edge_gather()
_TC_IN = _build_tc_in()
_TC_LAYER = _build_tc_layer()
_TC_AB = _build_tc_ab()
_TC_EDGE = _build_tc_edge()


def kernel(x, edge_index, edge_attr, Win_w, Win_b, Wl, bl, Wr, gam, bet,
           E1w, E1b, E2w, E2b, E3w, E3b):
    src = edge_index[0]
    dst = edge_index[1]
    pe = _EP - _E
    src_g = jnp.concatenate([src, jnp.zeros((pe,), jnp.int32)])
    dst_g = jnp.concatenate([dst, jnp.zeros((pe,), jnp.int32)])
    dst_s = jnp.concatenate([dst, jnp.full((pe,), _N, jnp.int32)])
    x_pad = jnp.concatenate([x, jnp.zeros((_NP - _N, _D), jnp.float32)])
    ea_pad = jnp.concatenate([edge_attr, jnp.zeros((pe, _DE), jnp.float32)])

    h = _TC_IN(x_pad, Win_w, Win_b.reshape(1, _H))
    degp = _SC_DEG(dst_s)
    for i in range(4):
        aggp = _SC_AGG(h, src_g, dst_s)
        h = _TC_LAYER(aggp, degp, h, Wl[i], bl[i].reshape(1, _H), Wr[i],
                      gam[i].reshape(1, _H), bet[i].reshape(1, _H))
    a, b = _TC_AB(h, E1w[:_H], E1w[_H:2 * _H])
    sa, sb = _SC_EDGE_GATHER(a, b, src_g, dst_g)
    out = _TC_EDGE(sa, sb, ea_pad, E1w[2 * _H:], E1b.reshape(1, _H),
                   E2w, E2b.reshape(1, 32), E3w.reshape(1, 32),
                   E3b.reshape(1, 1))
    return out[:_E, 0]


# speedup vs baseline: 3.4056x; 1.2662x over previous
import functools

import jax
import jax.numpy as jnp
from jax import lax
from jax.experimental import pallas as pl
from jax.experimental.pallas import tpu as pltpu
from jax.experimental.pallas import tpu_sc as plsc

_N = 10000
_NP = 10112
_E = 320000
_EP = 327680
_D = 128
_H = 64
_DE = 16
_NC = 2
_NS = 16
_K = 128
_NW = _NC * _NS
_EPW = _EP // _NW
_CH = _EPW // _K
_RPS = _NP // _NS


def _sc_mesh():
    return plsc.VectorSubcoreMesh(core_axis_name="c", subcore_axis_name="s")


_SC_PARAMS = pltpu.CompilerParams(use_tc_tiling_on_sc=False)


def _fill_const(ref, rows, width, value):
    v = jnp.full((16,), value, jnp.float32)

    def row(r, carry):
        def col(q, carry2):
            ref[r, pl.ds(q * 16, 16)] = v
            return carry2
        return lax.fori_loop(0, width // 16, col, carry)

    lax.fori_loop(0, rows, row, None)


def _zero_shared_slice(zsrc, shared, base, rows):
    full, rem = rows // _K, rows % _K
    for i in range(full):
        pltpu.sync_copy(zsrc, shared.at[pl.ds(base + i * _K, _K)])
    if rem:
        pltpu.sync_copy(zsrc.at[pl.ds(0, rem)], shared.at[pl.ds(base + full * _K, rem)])


def _build_sc_agg(with_deg):
    out_agg = jax.ShapeDtypeStruct((_NC, _NP, _H), jnp.float32)
    out_deg = jax.ShapeDtypeStruct((_NC, _NP, 16), jnp.float32)
    out_type = (out_agg, out_deg) if with_deg else out_agg
    scratch = [
        pltpu.VMEM((_CH, _K), jnp.int32),
        pltpu.VMEM((_CH, _K), jnp.int32),
        pltpu.VMEM((_K, _H), jnp.float32),
        pltpu.VMEM((_K, _H), jnp.float32),
        pltpu.VMEM((_K, _H), jnp.float32),
        pltpu.VMEM_SHARED((_NP, _H), jnp.float32),
        pltpu.SemaphoreType.DMA,
        pltpu.SemaphoreType.DMA,
    ]
    if with_deg:
        scratch += [
            pltpu.VMEM((_K, 16), jnp.float32),
            pltpu.VMEM((_K, 16), jnp.float32),
            pltpu.VMEM_SHARED((_NP, 16), jnp.float32),
        ]

    @functools.partial(
        pl.kernel,
        out_type=out_type,
        mesh=_sc_mesh(),
        compiler_params=_SC_PARAMS,
        scratch_types=scratch,
    )
    def agg_kernel(h_hbm, src_hbm, dst_hbm, *rest):
        if with_deg:
            (out_hbm, deg_hbm, src_v, dst_v, buf0, buf1, zer_v, agg_sh,
             sem0, sem1, ones_v, zer16_v, deg_sh) = rest
        else:
            (out_hbm, src_v, dst_v, buf0, buf1, zer_v, agg_sh,
             sem0, sem1) = rest
        c = lax.axis_index("c")
        s = lax.axis_index("s")
        wid = c * _NS + s
        _fill_const(zer_v, _K, _H, 0.0)
        base = s * _RPS
        _zero_shared_slice(zer_v, agg_sh, base, _RPS)
        if with_deg:
            _fill_const(ones_v, _K, 16, 1.0)
            _fill_const(zer16_v, _K, 16, 0.0)
            _zero_shared_slice(zer16_v, deg_sh, base, _RPS)
        plsc.subcore_barrier()

        pltpu.sync_copy(src_hbm.at[pl.ds(wid * _CH, _CH)], src_v)
        pltpu.sync_copy(dst_hbm.at[pl.ds(wid * _CH, _CH)], dst_v)

        def gstart(j, buf, sem):
            return pltpu.async_copy(h_hbm.at[src_v.at[j]], buf, sem)

        def gwait(buf, sem):
            pltpu.make_async_copy(h_hbm.at[src_v.at[0]], buf, sem).wait()

        def put(j, buf):
            pltpu.sync_copy(buf, agg_sh.at[dst_v.at[j]], add=True)
            if with_deg:
                pltpu.sync_copy(ones_v, deg_sh.at[dst_v.at[j]], add=True)

        gstart(0, buf0, sem0)

        def pair(p, carry):
            j0 = 2 * p
            gstart(j0 + 1, buf1, sem1)
            gwait(buf0, sem0)
            put(j0, buf0)
            jn = jnp.minimum(j0 + 2, _CH - 1)
            gstart(jn, buf0, sem0)
            gwait(buf1, sem1)
            put(j0 + 1, buf1)
            return carry

        lax.fori_loop(0, _CH // 2, pair, None)
        gwait(buf0, sem0)

        plsc.subcore_barrier()
        pltpu.sync_copy(agg_sh.at[pl.ds(base, _RPS)], out_hbm.at[c, pl.ds(base, _RPS)])
        if with_deg:
            pltpu.sync_copy(deg_sh.at[pl.ds(base, _RPS)], deg_hbm.at[c, pl.ds(base, _RPS)])

    return agg_kernel


def _build_sc_edge_gather():
    @functools.partial(
        pl.kernel,
        out_type=(jax.ShapeDtypeStruct((_EP, _H), jnp.float32),
                  jax.ShapeDtypeStruct((_EP, _H), jnp.float32)),
        mesh=_sc_mesh(),
        compiler_params=_SC_PARAMS,
        scratch_types=[
            pltpu.VMEM((_CH, _K), jnp.int32),
            pltpu.VMEM((_CH, _K), jnp.int32),
            pltpu.VMEM((_K, _H), jnp.float32),
            pltpu.VMEM((_K, _H), jnp.float32),
            pltpu.VMEM((_K, _H), jnp.float32),
            pltpu.VMEM((_K, _H), jnp.float32),
            pltpu.SemaphoreType.DMA,
            pltpu.SemaphoreType.DMA,
            pltpu.SemaphoreType.DMA,
            pltpu.SemaphoreType.DMA,
        ],
    )
    def gather_kernel(a_hbm, b_hbm, src_hbm, dst_hbm, sa_hbm, sb_hbm,
                      src_v, dst_v, a0, a1, b0, b1,
                      sema0, sema1, semb0, semb1):
        c = lax.axis_index("c")
        s = lax.axis_index("s")
        wid = c * _NS + s
        ebase = wid * _EPW
        pltpu.sync_copy(src_hbm.at[pl.ds(wid * _CH, _CH)], src_v)
        pltpu.sync_copy(dst_hbm.at[pl.ds(wid * _CH, _CH)], dst_v)

        def gstart(j, buf, sem, tab, idx):
            return pltpu.async_copy(tab.at[idx.at[j]], buf, sem)

        def gwait(buf, sem, tab, idx):
            pltpu.make_async_copy(tab.at[idx.at[0]], buf, sem).wait()

        def put(j, abuf, bbuf):
            o = ebase + j * _K
            pltpu.sync_copy(abuf, sa_hbm.at[pl.ds(o, _K)])
            pltpu.sync_copy(bbuf, sb_hbm.at[pl.ds(o, _K)])

        gstart(0, a0, sema0, a_hbm, src_v)
        gstart(0, b0, semb0, b_hbm, dst_v)

        def pair(p, carry):
            j0 = 2 * p
            gstart(j0 + 1, a1, sema1, a_hbm, src_v)
            gstart(j0 + 1, b1, semb1, b_hbm, dst_v)
            gwait(a0, sema0, a_hbm, src_v)
            gwait(b0, semb0, b_hbm, dst_v)
            put(j0, a0, b0)
            jn = jnp.minimum(j0 + 2, _CH - 1)
            gstart(jn, a0, sema0, a_hbm, src_v)
            gstart(jn, b0, semb0, b_hbm, dst_v)
            gwait(a1, sema1, a_hbm, src_v)
            gwait(b1, semb1, b_hbm, dst_v)
            put(j0 + 1, a1, b1)
            return carry

        lax.fori_loop(0, _CH // 2, pair, None)
        gwait(a0, sema0, a_hbm, src_v)
        gwait(b0, semb0, b_hbm, dst_v)

    return gather_kernel


def _tc_in_body(x_ref, w_ref, b_ref, o_ref):
    o_ref[...] = (jnp.dot(x_ref[...], w_ref[...],
                          preferred_element_type=jnp.float32) + b_ref[...])


def _build_tc_in():
    return pl.pallas_call(
        _tc_in_body,
        out_shape=jax.ShapeDtypeStruct((_NP, _H), jnp.float32),
    )


def _tc_layer_body(aggp_ref, degp_ref, h_ref, wl_ref, bl_ref, wr_ref,
                   g_ref, be_ref, o_ref):
    agg = aggp_ref[0] + aggp_ref[1]
    d = degp_ref[0] + degp_ref[1]
    deg = jnp.maximum(d[:, 0:1], 1.0)
    aggm = agg / deg
    h = h_ref[...]
    t = (jnp.dot(aggm, wl_ref[...], preferred_element_type=jnp.float32)
         + bl_ref[...]
         + jnp.dot(h, wr_ref[...], preferred_element_type=jnp.float32))
    mu = jnp.mean(t, axis=-1, keepdims=True)
    var = jnp.mean((t - mu) ** 2, axis=-1, keepdims=True)
    hn = (t - mu) * lax.rsqrt(var + 1e-5) * g_ref[...] + be_ref[...]
    o_ref[...] = jnp.maximum(hn, 0.0)


def _build_tc_layer():
    return pl.pallas_call(
        _tc_layer_body,
        out_shape=jax.ShapeDtypeStruct((_NP, _H), jnp.float32),
    )


def _tc_ab_body(h_ref, wu_ref, wv_ref, a_ref, b_ref):
    h = h_ref[...]
    a_ref[...] = jnp.dot(h, wu_ref[...], preferred_element_type=jnp.float32)
    b_ref[...] = jnp.dot(h, wv_ref[...], preferred_element_type=jnp.float32)


def _build_tc_ab():
    return pl.pallas_call(
        _tc_ab_body,
        out_shape=(jax.ShapeDtypeStruct((_NP, _H), jnp.float32),
                   jax.ShapeDtypeStruct((_NP, _H), jnp.float32)),
    )


_EB = 10240


def _tc_edge_body(sa_ref, sb_ref, ea_ref, w1_ref, b1_ref, w2_ref, b2_ref,
                  w3_ref, b3_ref, o_ref):
    z1 = jnp.maximum(
        sa_ref[...] + sb_ref[...]
        + jnp.dot(ea_ref[...], w1_ref[...], preferred_element_type=jnp.float32)
        + b1_ref[...], 0.0)
    z2 = jnp.maximum(
        jnp.dot(z1, w2_ref[...], preferred_element_type=jnp.float32)
        + b2_ref[...], 0.0)
    o_ref[...] = jnp.sum(z2 * w3_ref[...], axis=1, keepdims=True) + b3_ref[...]


def _build_tc_edge():
    return pl.pallas_call(
        _tc_edge_body,
        grid=(_EP // _EB,),
        in_specs=[
            pl.BlockSpec((_EB, _H), lambda i: (i, 0)),
            pl.BlockSpec((_EB, _H), lambda i: (i, 0)),
            pl.BlockSpec((_EB, _DE), lambda i: (i, 0)),
            pl.BlockSpec((_DE, _H), lambda i: (0, 0)),
            pl.BlockSpec((1, _H), lambda i: (0, 0)),
            pl.BlockSpec((_H, 32), lambda i: (0, 0)),
            pl.BlockSpec((1, 32), lambda i: (0, 0)),
            pl.BlockSpec((1, 32), lambda i: (0, 0)),
            pl.BlockSpec((1, 1), lambda i: (0, 0)),
        ],
        out_specs=pl.BlockSpec((_EB, 1), lambda i: (i, 0)),
        out_shape=jax.ShapeDtypeStruct((_EP, 1), jnp.float32),
    )


_SC_AGG_DEG = _build_sc_agg(True)
_SC_AGG = _build_sc_agg(False)
_SC_EDGE_GATHER = _build_sc_edge_gather()
_TC_IN = _build_tc_in()
_TC_LAYER = _build_tc_layer()
_TC_AB = _build_tc_ab()
_TC_EDGE = _build_tc_edge()


def kernel(x, edge_index, edge_attr, Win_w, Win_b, Wl, bl, Wr, gam, bet,
           E1w, E1b, E2w, E2b, E3w, E3b):
    src = edge_index[0]
    dst = edge_index[1]
    pe = _EP - _E
    src_g = jnp.concatenate([src, jnp.zeros((pe,), jnp.int32)]).reshape(_NW * _CH, _K)
    dst_g = jnp.concatenate([dst, jnp.zeros((pe,), jnp.int32)]).reshape(_NW * _CH, _K)
    dst_s = jnp.concatenate([dst, jnp.full((pe,), _N, jnp.int32)]).reshape(_NW * _CH, _K)
    x_pad = jnp.concatenate([x, jnp.zeros((_NP - _N, _D), jnp.float32)])
    ea_pad = jnp.concatenate([edge_attr, jnp.zeros((pe, _DE), jnp.float32)])

    h = _TC_IN(x_pad, Win_w, Win_b.reshape(1, _H))
    for i in range(4):
        if i == 0:
            aggp, degp = _SC_AGG_DEG(h, src_g, dst_s)
        else:
            aggp = _SC_AGG(h, src_g, dst_s)
        h = _TC_LAYER(aggp, degp, h, Wl[i], bl[i].reshape(1, _H), Wr[i],
                      gam[i].reshape(1, _H), bet[i].reshape(1, _H))
    a, b = _TC_AB(h, E1w[:_H], E1w[_H:2 * _H])
    sa, sb = _SC_EDGE_GATHER(a, b, src_g, dst_g)
    out = _TC_EDGE(sa, sb, ea_pad, E1w[2 * _H:], E1b.reshape(1, _H),
                   E2w, E2b.reshape(1, 32), E3w.reshape(1, 32),
                   E3b.reshape(1, 1))
    return out[:_E, 0]


# speedup vs baseline: 3.6592x; 1.0745x over previous
import functools

import jax
import jax.numpy as jnp
from jax import lax
from jax.experimental import pallas as pl
from jax.experimental.pallas import tpu as pltpu
from jax.experimental.pallas import tpu_sc as plsc

_N = 10000
_NP = 10112
_E = 320000
_EP = 327680
_D = 128
_H = 64
_DE = 16
_NC = 2
_NS = 16
_K = 128
_NW = _NC * _NS
_CH = _EP // (_NW * _K)
_CH0 = 120
_CH1 = 2 * _CH - _CH0
_CHMAX = max(_CH0, _CH1)
_IDXROWS = _NW * _CH + _CHMAX
_RPS = _NP // _NS


def _sc_mesh():
    return plsc.VectorSubcoreMesh(core_axis_name="c", subcore_axis_name="s")


_SC_PARAMS = pltpu.CompilerParams(use_tc_tiling_on_sc=False)


def _fill_const(ref, rows, width, value):
    v = jnp.full((16,), value, jnp.float32)

    def row(r, carry):
        def col(q, carry2):
            ref[r, pl.ds(q * 16, 16)] = v
            return carry2
        return lax.fori_loop(0, width // 16, col, carry)

    lax.fori_loop(0, rows, row, None)


def _zero_shared_slice(zsrc, shared, base, rows):
    full, rem = rows // _K, rows % _K
    for i in range(full):
        pltpu.sync_copy(zsrc, shared.at[pl.ds(base + i * _K, _K)])
    if rem:
        pltpu.sync_copy(zsrc.at[pl.ds(0, rem)], shared.at[pl.ds(base + full * _K, rem)])


def _build_sc_agg(with_deg):
    out_agg = jax.ShapeDtypeStruct((_NC, _NP, _H), jnp.float32)
    out_deg = jax.ShapeDtypeStruct((_NC, _NP, 16), jnp.float32)
    out_type = (out_agg, out_deg) if with_deg else out_agg
    scratch = [
        pltpu.VMEM((_CHMAX, _K), jnp.int32),
        pltpu.VMEM((_CHMAX, _K), jnp.int32),
        pltpu.VMEM((_K, _H), jnp.float32),
        pltpu.VMEM((_K, _H), jnp.float32),
        pltpu.VMEM((_K, _H), jnp.float32),
        pltpu.VMEM_SHARED((_NP, _H), jnp.float32),
        pltpu.SemaphoreType.DMA,
        pltpu.SemaphoreType.DMA,
    ]
    if with_deg:
        scratch += [
            pltpu.VMEM((_K, 16), jnp.float32),
            pltpu.VMEM((_K, 16), jnp.float32),
            pltpu.VMEM_SHARED((_NP, 16), jnp.float32),
        ]

    @functools.partial(
        pl.kernel,
        out_type=out_type,
        mesh=_sc_mesh(),
        compiler_params=_SC_PARAMS,
        scratch_types=scratch,
    )
    def agg_kernel(h_hbm, src_hbm, dst_hbm, *rest):
        if with_deg:
            (out_hbm, deg_hbm, src_v, dst_v, buf0, buf1, zer_v, agg_sh,
             sem0, sem1, ones_v, zer16_v, deg_sh) = rest
        else:
            (out_hbm, src_v, dst_v, buf0, buf1, zer_v, agg_sh,
             sem0, sem1) = rest
        c = lax.axis_index("c")
        s = lax.axis_index("s")
        rb = jnp.where(c == 0, s * _CH0, _NS * _CH0 + s * _CH1)
        nch = jnp.where(c == 0, _CH0, _CH1)
        trips = jnp.where(c == 0, _CH0 // 2, _CH1 // 2)
        _fill_const(zer_v, _K, _H, 0.0)
        base = s * _RPS
        _zero_shared_slice(zer_v, agg_sh, base, _RPS)
        if with_deg:
            _fill_const(ones_v, _K, 16, 1.0)
            _fill_const(zer16_v, _K, 16, 0.0)
            _zero_shared_slice(zer16_v, deg_sh, base, _RPS)
        plsc.subcore_barrier()

        pltpu.sync_copy(src_hbm.at[pl.ds(rb, _CHMAX)], src_v)
        pltpu.sync_copy(dst_hbm.at[pl.ds(rb, _CHMAX)], dst_v)

        def gstart(j, buf, sem):
            return pltpu.async_copy(h_hbm.at[src_v.at[j]], buf, sem)

        def gwait(buf, sem):
            pltpu.make_async_copy(h_hbm.at[src_v.at[0]], buf, sem).wait()

        def put(j, buf):
            pltpu.sync_copy(buf, agg_sh.at[dst_v.at[j]], add=True)
            if with_deg:
                pltpu.sync_copy(ones_v, deg_sh.at[dst_v.at[j]], add=True)

        gstart(0, buf0, sem0)

        def pair(p, carry):
            j0 = 2 * p
            gstart(j0 + 1, buf1, sem1)
            gwait(buf0, sem0)
            put(j0, buf0)
            jn = jnp.minimum(j0 + 2, nch - 1)
            gstart(jn, buf0, sem0)
            gwait(buf1, sem1)
            put(j0 + 1, buf1)
            return carry

        lax.fori_loop(0, trips, pair, None)
        gwait(buf0, sem0)

        plsc.subcore_barrier()
        pltpu.sync_copy(agg_sh.at[pl.ds(base, _RPS)], out_hbm.at[c, pl.ds(base, _RPS)])
        if with_deg:
            pltpu.sync_copy(deg_sh.at[pl.ds(base, _RPS)], deg_hbm.at[c, pl.ds(base, _RPS)])

    return agg_kernel


def _build_sc_edge_gather():
    @functools.partial(
        pl.kernel,
        out_type=(jax.ShapeDtypeStruct((_EP, _H), jnp.float32),
                  jax.ShapeDtypeStruct((_EP, _H), jnp.float32)),
        mesh=_sc_mesh(),
        compiler_params=_SC_PARAMS,
        scratch_types=[
            pltpu.VMEM((_CHMAX, _K), jnp.int32),
            pltpu.VMEM((_CHMAX, _K), jnp.int32),
            pltpu.VMEM((_K, _H), jnp.float32),
            pltpu.VMEM((_K, _H), jnp.float32),
            pltpu.VMEM((_K, _H), jnp.float32),
            pltpu.VMEM((_K, _H), jnp.float32),
            pltpu.SemaphoreType.DMA,
            pltpu.SemaphoreType.DMA,
            pltpu.SemaphoreType.DMA,
            pltpu.SemaphoreType.DMA,
        ],
    )
    def gather_kernel(a_hbm, b_hbm, src_hbm, dst_hbm, sa_hbm, sb_hbm,
                      src_v, dst_v, a0, a1, b0, b1,
                      sema0, sema1, semb0, semb1):
        c = lax.axis_index("c")
        s = lax.axis_index("s")
        rb = jnp.where(c == 0, s * _CH0, _NS * _CH0 + s * _CH1)
        nch = jnp.where(c == 0, _CH0, _CH1)
        trips = jnp.where(c == 0, _CH0 // 2, _CH1 // 2)
        pltpu.sync_copy(src_hbm.at[pl.ds(rb, _CHMAX)], src_v)
        pltpu.sync_copy(dst_hbm.at[pl.ds(rb, _CHMAX)], dst_v)

        def gstart(j, buf, sem, tab, idx):
            return pltpu.async_copy(tab.at[idx.at[j]], buf, sem)

        def gwait(buf, sem, tab, idx):
            pltpu.make_async_copy(tab.at[idx.at[0]], buf, sem).wait()

        def put(j, abuf, bbuf):
            o = (rb + j) * _K
            pltpu.sync_copy(abuf, sa_hbm.at[pl.ds(o, _K)])
            pltpu.sync_copy(bbuf, sb_hbm.at[pl.ds(o, _K)])

        gstart(0, a0, sema0, a_hbm, src_v)
        gstart(0, b0, semb0, b_hbm, dst_v)

        def pair(p, carry):
            j0 = 2 * p
            gstart(j0 + 1, a1, sema1, a_hbm, src_v)
            gstart(j0 + 1, b1, semb1, b_hbm, dst_v)
            gwait(a0, sema0, a_hbm, src_v)
            gwait(b0, semb0, b_hbm, dst_v)
            put(j0, a0, b0)
            jn = jnp.minimum(j0 + 2, nch - 1)
            gstart(jn, a0, sema0, a_hbm, src_v)
            gstart(jn, b0, semb0, b_hbm, dst_v)
            gwait(a1, sema1, a_hbm, src_v)
            gwait(b1, semb1, b_hbm, dst_v)
            put(j0 + 1, a1, b1)
            return carry

        lax.fori_loop(0, trips, pair, None)
        gwait(a0, sema0, a_hbm, src_v)
        gwait(b0, semb0, b_hbm, dst_v)

    return gather_kernel


def _tc_in_body(x_ref, w_ref, b_ref, o_ref):
    o_ref[...] = (jnp.dot(x_ref[...], w_ref[...],
                          preferred_element_type=jnp.float32) + b_ref[...])


def _build_tc_in():
    return pl.pallas_call(
        _tc_in_body,
        out_shape=jax.ShapeDtypeStruct((_NP, _H), jnp.float32),
    )


def _tc_layer_body(aggp_ref, degp_ref, h_ref, wl_ref, bl_ref, wr_ref,
                   g_ref, be_ref, o_ref):
    agg = aggp_ref[0] + aggp_ref[1]
    d = degp_ref[0] + degp_ref[1]
    deg = jnp.maximum(d[:, 0:1], 1.0)
    aggm = agg / deg
    h = h_ref[...]
    t = (jnp.dot(aggm, wl_ref[...], preferred_element_type=jnp.float32)
         + bl_ref[...]
         + jnp.dot(h, wr_ref[...], preferred_element_type=jnp.float32))
    mu = jnp.mean(t, axis=-1, keepdims=True)
    var = jnp.mean((t - mu) ** 2, axis=-1, keepdims=True)
    hn = (t - mu) * lax.rsqrt(var + 1e-5) * g_ref[...] + be_ref[...]
    o_ref[...] = jnp.maximum(hn, 0.0)


def _build_tc_layer():
    return pl.pallas_call(
        _tc_layer_body,
        out_shape=jax.ShapeDtypeStruct((_NP, _H), jnp.float32),
    )


def _tc_ab_body(h_ref, wu_ref, wv_ref, a_ref, b_ref):
    h = h_ref[...]
    a_ref[...] = jnp.dot(h, wu_ref[...], preferred_element_type=jnp.float32)
    b_ref[...] = jnp.dot(h, wv_ref[...], preferred_element_type=jnp.float32)


def _build_tc_ab():
    return pl.pallas_call(
        _tc_ab_body,
        out_shape=(jax.ShapeDtypeStruct((_NP, _H), jnp.float32),
                   jax.ShapeDtypeStruct((_NP, _H), jnp.float32)),
    )


_EB = 10240


def _tc_edge_body(sa_ref, sb_ref, ea_ref, w1_ref, b1_ref, w2_ref, b2_ref,
                  w3_ref, b3_ref, o_ref):
    z1 = jnp.maximum(
        sa_ref[...] + sb_ref[...]
        + jnp.dot(ea_ref[...], w1_ref[...], preferred_element_type=jnp.float32)
        + b1_ref[...], 0.0)
    z2 = jnp.maximum(
        jnp.dot(z1, w2_ref[...], preferred_element_type=jnp.float32)
        + b2_ref[...], 0.0)
    o_ref[...] = jnp.sum(z2 * w3_ref[...], axis=1, keepdims=True) + b3_ref[...]


def _build_tc_edge():
    return pl.pallas_call(
        _tc_edge_body,
        grid=(_EP // _EB,),
        in_specs=[
            pl.BlockSpec((_EB, _H), lambda i: (i, 0)),
            pl.BlockSpec((_EB, _H), lambda i: (i, 0)),
            pl.BlockSpec((_EB, _DE), lambda i: (i, 0)),
            pl.BlockSpec((_DE, _H), lambda i: (0, 0)),
            pl.BlockSpec((1, _H), lambda i: (0, 0)),
            pl.BlockSpec((_H, 32), lambda i: (0, 0)),
            pl.BlockSpec((1, 32), lambda i: (0, 0)),
            pl.BlockSpec((1, 32), lambda i: (0, 0)),
            pl.BlockSpec((1, 1), lambda i: (0, 0)),
        ],
        out_specs=pl.BlockSpec((_EB, 1), lambda i: (i, 0)),
        out_shape=jax.ShapeDtypeStruct((_EP, 1), jnp.float32),
    )


_SC_AGG_DEG = _build_sc_agg(True)
_SC_AGG = _build_sc_agg(False)
_SC_EDGE_GATHER = _build_sc_edge_gather()
_TC_IN = _build_tc_in()
_TC_LAYER = _build_tc_layer()
_TC_AB = _build_tc_ab()
_TC_EDGE = _build_tc_edge()


def kernel(x, edge_index, edge_attr, Win_w, Win_b, Wl, bl, Wr, gam, bet,
           E1w, E1b, E2w, E2b, E3w, E3b):
    src = edge_index[0]
    dst = edge_index[1]
    pe = _EP - _E
    tail = _CHMAX * _K
    src_g = jnp.concatenate([src, jnp.zeros((pe + tail,), jnp.int32)]).reshape(_IDXROWS, _K)
    dst_g = jnp.concatenate([dst, jnp.zeros((pe + tail,), jnp.int32)]).reshape(_IDXROWS, _K)
    dst_s = jnp.concatenate([dst, jnp.full((pe,), _N, jnp.int32),
                             jnp.zeros((tail,), jnp.int32)]).reshape(_IDXROWS, _K)
    x_pad = jnp.concatenate([x, jnp.zeros((_NP - _N, _D), jnp.float32)])
    ea_pad = jnp.concatenate([edge_attr, jnp.zeros((pe, _DE), jnp.float32)])

    h = _TC_IN(x_pad, Win_w, Win_b.reshape(1, _H))
    for i in range(4):
        if i == 0:
            aggp, degp = _SC_AGG_DEG(h, src_g, dst_s)
        else:
            aggp = _SC_AGG(h, src_g, dst_s)
        h = _TC_LAYER(aggp, degp, h, Wl[i], bl[i].reshape(1, _H), Wr[i],
                      gam[i].reshape(1, _H), bet[i].reshape(1, _H))
    a, b = _TC_AB(h, E1w[:_H], E1w[_H:2 * _H])
    sa, sb = _SC_EDGE_GATHER(a, b, src_g, dst_g)
    out = _TC_EDGE(sa, sb, ea_pad, E1w[2 * _H:], E1b.reshape(1, _H),
                   E2w, E2b.reshape(1, 32), E3w.reshape(1, 32),
                   E3b.reshape(1, 1))
    return out[:_E, 0]


# speedup vs baseline: 3.6618x; 1.0007x over previous
import functools

import jax
import jax.numpy as jnp
from jax import lax
from jax.experimental import pallas as pl
from jax.experimental.pallas import tpu as pltpu
from jax.experimental.pallas import tpu_sc as plsc

_N = 10000
_NP = 10112
_E = 320000
_EP = 327680
_D = 128
_H = 64
_DE = 16
_NC = 2
_NS = 16
_K = 128
_NW = _NC * _NS
_CH = _EP // (_NW * _K)
_CH0 = 120
_CH1 = 2 * _CH - _CH0
_CHMAX = max(_CH0, _CH1)
_IDXROWS = _NW * _CH + _CHMAX
_RPS = _NP // _NS


def _sc_mesh():
    return plsc.VectorSubcoreMesh(core_axis_name="c", subcore_axis_name="s")


_SC_PARAMS = pltpu.CompilerParams(use_tc_tiling_on_sc=False)


def _fill_const(ref, rows, width, value):
    v = jnp.full((16,), value, jnp.float32)

    def row(r, carry):
        def col(q, carry2):
            ref[r, pl.ds(q * 16, 16)] = v
            return carry2
        return lax.fori_loop(0, width // 16, col, carry)

    lax.fori_loop(0, rows, row, None)


def _zero_shared_slice(zsrc, shared, base, rows):
    full, rem = rows // _K, rows % _K
    for i in range(full):
        pltpu.sync_copy(zsrc, shared.at[pl.ds(base + i * _K, _K)])
    if rem:
        pltpu.sync_copy(zsrc.at[pl.ds(0, rem)], shared.at[pl.ds(base + full * _K, rem)])


def _build_sc_agg(with_deg):
    out_agg = jax.ShapeDtypeStruct((_NC, _NP, _H), jnp.float32)
    out_deg = jax.ShapeDtypeStruct((_NC, _NP, 16), jnp.float32)
    out_type = (out_agg, out_deg) if with_deg else out_agg
    scratch = [
        pltpu.VMEM((_CHMAX, _K), jnp.int32),
        pltpu.VMEM((_CHMAX, _K), jnp.int32),
        pltpu.VMEM((_K, _H), jnp.float32),
        pltpu.VMEM((_K, _H), jnp.float32),
        pltpu.VMEM((_K, _H), jnp.float32),
        pltpu.VMEM_SHARED((_NP, _H), jnp.float32),
        pltpu.SemaphoreType.DMA,
        pltpu.SemaphoreType.DMA,
    ]
    if with_deg:
        scratch += [
            pltpu.VMEM((_K, 16), jnp.float32),
            pltpu.VMEM((_K, 16), jnp.float32),
            pltpu.VMEM_SHARED((_NP, 16), jnp.float32),
        ]

    @functools.partial(
        pl.kernel,
        out_type=out_type,
        mesh=_sc_mesh(),
        compiler_params=_SC_PARAMS,
        scratch_types=scratch,
    )
    def agg_kernel(h_hbm, src_hbm, dst_hbm, *rest):
        if with_deg:
            (out_hbm, deg_hbm, src_v, dst_v, buf0, buf1, zer_v, agg_sh,
             sem0, sem1, ones_v, zer16_v, deg_sh) = rest
        else:
            (out_hbm, src_v, dst_v, buf0, buf1, zer_v, agg_sh,
             sem0, sem1) = rest
        c = lax.axis_index("c")
        s = lax.axis_index("s")
        rb = jnp.where(c == 0, s * _CH0, _NS * _CH0 + s * _CH1)
        nch = jnp.where(c == 0, _CH0, _CH1)
        trips = jnp.where(c == 0, _CH0 // 2, _CH1 // 2)
        with jax.named_scope("ph_zero"):
            _fill_const(zer_v, _K, _H, 0.0)
            base = s * _RPS
            _zero_shared_slice(zer_v, agg_sh, base, _RPS)
            if with_deg:
                _fill_const(ones_v, _K, 16, 1.0)
                _fill_const(zer16_v, _K, 16, 0.0)
                _zero_shared_slice(zer16_v, deg_sh, base, _RPS)
        with jax.named_scope("ph_barrier"):
            plsc.subcore_barrier()

        with jax.named_scope("ph_hoist"):
            pltpu.sync_copy(src_hbm.at[pl.ds(rb, _CHMAX)], src_v)
            pltpu.sync_copy(dst_hbm.at[pl.ds(rb, _CHMAX)], dst_v)

        def gstart(j, buf, sem):
            return pltpu.async_copy(h_hbm.at[src_v.at[j]], buf, sem)

        def gwait(buf, sem):
            pltpu.make_async_copy(h_hbm.at[src_v.at[0]], buf, sem).wait()

        def put(j, buf):
            pltpu.sync_copy(buf, agg_sh.at[dst_v.at[j]], add=True)
            if with_deg:
                pltpu.sync_copy(ones_v, deg_sh.at[dst_v.at[j]], add=True)

        gstart(0, buf0, sem0)

        def pair(p, carry):
            j0 = 2 * p
            gstart(j0 + 1, buf1, sem1)
            gwait(buf0, sem0)
            put(j0, buf0)
            jn = jnp.minimum(j0 + 2, nch - 1)
            gstart(jn, buf0, sem0)
            gwait(buf1, sem1)
            put(j0 + 1, buf1)
            return carry

        with jax.named_scope("ph_loop"):
            lax.fori_loop(0, trips, pair, None)
            gwait(buf0, sem0)

        with jax.named_scope("ph_barrier2"):
            plsc.subcore_barrier()
        with jax.named_scope("ph_out"):
            pltpu.sync_copy(agg_sh.at[pl.ds(base, _RPS)], out_hbm.at[c, pl.ds(base, _RPS)])
            if with_deg:
                pltpu.sync_copy(deg_sh.at[pl.ds(base, _RPS)], deg_hbm.at[c, pl.ds(base, _RPS)])

    return agg_kernel


def _build_sc_edge_gather():
    @functools.partial(
        pl.kernel,
        out_type=(jax.ShapeDtypeStruct((_EP, _H), jnp.float32),
                  jax.ShapeDtypeStruct((_EP, _H), jnp.float32)),
        mesh=_sc_mesh(),
        compiler_params=_SC_PARAMS,
        scratch_types=[
            pltpu.VMEM((_CHMAX, _K), jnp.int32),
            pltpu.VMEM((_CHMAX, _K), jnp.int32),
            pltpu.VMEM((_K, _H), jnp.float32),
            pltpu.VMEM((_K, _H), jnp.float32),
            pltpu.VMEM((_K, _H), jnp.float32),
            pltpu.VMEM((_K, _H), jnp.float32),
            pltpu.SemaphoreType.DMA,
            pltpu.SemaphoreType.DMA,
            pltpu.SemaphoreType.DMA,
            pltpu.SemaphoreType.DMA,
        ],
    )
    def gather_kernel(a_hbm, b_hbm, src_hbm, dst_hbm, sa_hbm, sb_hbm,
                      src_v, dst_v, a0, a1, b0, b1,
                      sema0, sema1, semb0, semb1):
        c = lax.axis_index("c")
        s = lax.axis_index("s")
        rb = jnp.where(c == 0, s * _CH0, _NS * _CH0 + s * _CH1)
        nch = jnp.where(c == 0, _CH0, _CH1)
        trips = jnp.where(c == 0, _CH0 // 2, _CH1 // 2)
        pltpu.sync_copy(src_hbm.at[pl.ds(rb, _CHMAX)], src_v)
        pltpu.sync_copy(dst_hbm.at[pl.ds(rb, _CHMAX)], dst_v)

        def gstart(j, buf, sem, tab, idx):
            return pltpu.async_copy(tab.at[idx.at[j]], buf, sem)

        def gwait(buf, sem, tab, idx):
            pltpu.make_async_copy(tab.at[idx.at[0]], buf, sem).wait()

        def put(j, abuf, bbuf):
            o = (rb + j) * _K
            pltpu.sync_copy(abuf, sa_hbm.at[pl.ds(o, _K)])
            pltpu.sync_copy(bbuf, sb_hbm.at[pl.ds(o, _K)])

        gstart(0, a0, sema0, a_hbm, src_v)
        gstart(0, b0, semb0, b_hbm, dst_v)

        def pair(p, carry):
            j0 = 2 * p
            gstart(j0 + 1, a1, sema1, a_hbm, src_v)
            gstart(j0 + 1, b1, semb1, b_hbm, dst_v)
            gwait(a0, sema0, a_hbm, src_v)
            gwait(b0, semb0, b_hbm, dst_v)
            put(j0, a0, b0)
            jn = jnp.minimum(j0 + 2, nch - 1)
            gstart(jn, a0, sema0, a_hbm, src_v)
            gstart(jn, b0, semb0, b_hbm, dst_v)
            gwait(a1, sema1, a_hbm, src_v)
            gwait(b1, semb1, b_hbm, dst_v)
            put(j0 + 1, a1, b1)
            return carry

        lax.fori_loop(0, trips, pair, None)
        gwait(a0, sema0, a_hbm, src_v)
        gwait(b0, semb0, b_hbm, dst_v)

    return gather_kernel


def _tc_in_body(x_ref, w_ref, b_ref, o_ref):
    o_ref[...] = (jnp.dot(x_ref[...], w_ref[...],
                          preferred_element_type=jnp.float32) + b_ref[...])


def _build_tc_in():
    return pl.pallas_call(
        _tc_in_body,
        out_shape=jax.ShapeDtypeStruct((_NP, _H), jnp.float32),
    )


def _tc_layer_body(aggp_ref, degp_ref, h_ref, wl_ref, bl_ref, wr_ref,
                   g_ref, be_ref, o_ref):
    agg = aggp_ref[0] + aggp_ref[1]
    d = degp_ref[0] + degp_ref[1]
    deg = jnp.maximum(d[:, 0:1], 1.0)
    aggm = agg / deg
    h = h_ref[...]
    t = (jnp.dot(aggm, wl_ref[...], preferred_element_type=jnp.float32)
         + bl_ref[...]
         + jnp.dot(h, wr_ref[...], preferred_element_type=jnp.float32))
    mu = jnp.mean(t, axis=-1, keepdims=True)
    var = jnp.mean((t - mu) ** 2, axis=-1, keepdims=True)
    hn = (t - mu) * lax.rsqrt(var + 1e-5) * g_ref[...] + be_ref[...]
    o_ref[...] = jnp.maximum(hn, 0.0)


def _build_tc_layer():
    return pl.pallas_call(
        _tc_layer_body,
        out_shape=jax.ShapeDtypeStruct((_NP, _H), jnp.float32),
    )


def _tc_ab_body(h_ref, wu_ref, wv_ref, a_ref, b_ref):
    h = h_ref[...]
    a_ref[...] = jnp.dot(h, wu_ref[...], preferred_element_type=jnp.float32)
    b_ref[...] = jnp.dot(h, wv_ref[...], preferred_element_type=jnp.float32)


def _build_tc_ab():
    return pl.pallas_call(
        _tc_ab_body,
        out_shape=(jax.ShapeDtypeStruct((_NP, _H), jnp.float32),
                   jax.ShapeDtypeStruct((_NP, _H), jnp.float32)),
    )


_EB = 10240


def _tc_edge_body(sa_ref, sb_ref, ea_ref, w1_ref, b1_ref, w2_ref, b2_ref,
                  w3_ref, b3_ref, o_ref):
    z1 = jnp.maximum(
        sa_ref[...] + sb_ref[...]
        + jnp.dot(ea_ref[...], w1_ref[...], preferred_element_type=jnp.float32)
        + b1_ref[...], 0.0)
    z2 = jnp.maximum(
        jnp.dot(z1, w2_ref[...], preferred_element_type=jnp.float32)
        + b2_ref[...], 0.0)
    o_ref[...] = jnp.sum(z2 * w3_ref[...], axis=1, keepdims=True) + b3_ref[...]


def _build_tc_edge():
    return pl.pallas_call(
        _tc_edge_body,
        grid=(_EP // _EB,),
        in_specs=[
            pl.BlockSpec((_EB, _H), lambda i: (i, 0)),
            pl.BlockSpec((_EB, _H), lambda i: (i, 0)),
            pl.BlockSpec((_EB, _DE), lambda i: (i, 0)),
            pl.BlockSpec((_DE, _H), lambda i: (0, 0)),
            pl.BlockSpec((1, _H), lambda i: (0, 0)),
            pl.BlockSpec((_H, 32), lambda i: (0, 0)),
            pl.BlockSpec((1, 32), lambda i: (0, 0)),
            pl.BlockSpec((1, 32), lambda i: (0, 0)),
            pl.BlockSpec((1, 1), lambda i: (0, 0)),
        ],
        out_specs=pl.BlockSpec((_EB, 1), lambda i: (i, 0)),
        out_shape=jax.ShapeDtypeStruct((_EP, 1), jnp.float32),
    )


_SC_AGG_DEG = _build_sc_agg(True)
_SC_AGG = _build_sc_agg(False)
_SC_EDGE_GATHER = _build_sc_edge_gather()
_TC_IN = _build_tc_in()
_TC_LAYER = _build_tc_layer()
_TC_AB = _build_tc_ab()
_TC_EDGE = _build_tc_edge()


def kernel(x, edge_index, edge_attr, Win_w, Win_b, Wl, bl, Wr, gam, bet,
           E1w, E1b, E2w, E2b, E3w, E3b):
    src = edge_index[0]
    dst = edge_index[1]
    pe = _EP - _E
    tail = _CHMAX * _K
    src_g = jnp.concatenate([src, jnp.zeros((pe + tail,), jnp.int32)]).reshape(_IDXROWS, _K)
    dst_g = jnp.concatenate([dst, jnp.zeros((pe + tail,), jnp.int32)]).reshape(_IDXROWS, _K)
    dst_s = jnp.concatenate([dst, jnp.full((pe,), _N, jnp.int32),
                             jnp.zeros((tail,), jnp.int32)]).reshape(_IDXROWS, _K)
    x_pad = jnp.concatenate([x, jnp.zeros((_NP - _N, _D), jnp.float32)])
    ea_pad = jnp.concatenate([edge_attr, jnp.zeros((pe, _DE), jnp.float32)])

    h = _TC_IN(x_pad, Win_w, Win_b.reshape(1, _H))
    for i in range(4):
        if i == 0:
            aggp, degp = _SC_AGG_DEG(h, src_g, dst_s)
        else:
            aggp = _SC_AGG(h, src_g, dst_s)
        h = _TC_LAYER(aggp, degp, h, Wl[i], bl[i].reshape(1, _H), Wr[i],
                      gam[i].reshape(1, _H), bet[i].reshape(1, _H))
    a, b = _TC_AB(h, E1w[:_H], E1w[_H:2 * _H])
    sa, sb = _SC_EDGE_GATHER(a, b, src_g, dst_g)
    out = _TC_EDGE(sa, sb, ea_pad, E1w[2 * _H:], E1b.reshape(1, _H),
                   E2w, E2b.reshape(1, 32), E3w.reshape(1, 32),
                   E3b.reshape(1, 1))
    return out[:_E, 0]
